# Initial kernel scaffold; baseline (speedup 1.0000x reference)
import jax
import jax.numpy as jnp
from jax.experimental import pallas as pl


def kernel(item_idxs, user_items_flat, cu_seqlens, neighbor_table, feat_tables, W1a, b1a, W1b, b1b, W2, b2, W_int, b_int):
    raise NotImplementedError("write your pallas kernel here")


# speedup vs baseline: 1.0904x; 1.0904x over previous
import functools

import jax
import jax.numpy as jnp
from jax import lax
from jax.experimental import pallas as pl
from jax.experimental.pallas import tpu as pltpu
from jax.experimental.pallas import tpu_sc as plsc

N_ITEMS = 100000
IN_DIM = 64
OUT_DIM = 64
N_COM = 5
TOP_K = 8
FEAT = N_COM * OUT_DIM

NC = 2
NS = 16
NW = NC * NS

_Z_ROWS_BLK = 1000


def _zbuild_body(ft_ref, w1a_ref, b1a_ref, w1b_ref, b1b_ref, w2_ref, b2_ref, z_ref):
    for c in range(N_COM):
        xa = ft_ref[2 * c]
        xb = ft_ref[2 * c + 1]
        ha = jnp.maximum(
            jnp.dot(xa, w1a_ref[c], preferred_element_type=jnp.float32) + b1a_ref[c], 0.0)
        hb = jnp.maximum(
            jnp.dot(xb, w1b_ref[c], preferred_element_type=jnp.float32) + b1b_ref[c], 0.0)
        z = jnp.dot(ha + hb, w2_ref[c], preferred_element_type=jnp.float32) + b2_ref[c]
        z_ref[:, c * OUT_DIM:(c + 1) * OUT_DIM] = z


def _build_z(feat_tables, W1a, b1a, W1b, b1b, W2, b2):
    n = feat_tables.shape[1]
    grid = n // _Z_ROWS_BLK
    return pl.pallas_call(
        _zbuild_body,
        grid=(grid,),
        in_specs=[
            pl.BlockSpec((2 * N_COM, _Z_ROWS_BLK, IN_DIM), lambda i: (0, i, 0)),
            pl.BlockSpec((N_COM, IN_DIM, OUT_DIM), lambda i: (0, 0, 0)),
            pl.BlockSpec((N_COM, OUT_DIM), lambda i: (0, 0)),
            pl.BlockSpec((N_COM, IN_DIM, OUT_DIM), lambda i: (0, 0, 0)),
            pl.BlockSpec((N_COM, OUT_DIM), lambda i: (0, 0)),
            pl.BlockSpec((N_COM, OUT_DIM, OUT_DIM), lambda i: (0, 0, 0)),
            pl.BlockSpec((N_COM, OUT_DIM), lambda i: (0, 0)),
        ],
        out_specs=pl.BlockSpec((_Z_ROWS_BLK, FEAT), lambda i: (i, 0)),
        out_shape=jax.ShapeDtypeStruct((n, FEAT), jnp.float32),
    )(feat_tables, W1a, b1a, W1b, b1b, W2, b2)


def _nbr_gather_kernel(mp, idx_chunk, chunks_per_w):
    cpw = mp // NW
    mesh = plsc.VectorSubcoreMesh(core_axis_name="c", subcore_axis_name="s")

    @functools.partial(
        pl.kernel,
        out_type=jax.ShapeDtypeStruct((mp, TOP_K), jnp.int32),
        mesh=mesh,
        scratch_types=[
            pltpu.VMEM((chunks_per_w, idx_chunk), jnp.int32),
            pltpu.VMEM((cpw, TOP_K), jnp.int32),
            pltpu.SemaphoreType.DMA,
        ],
        compiler_params=pltpu.CompilerParams(use_tc_tiling_on_sc=False),
    )
    def k(table_hbm, idx_hbm, out_hbm, idx_v, rows_v, sem):
        wid = lax.axis_index("s") * NC + lax.axis_index("c")
        pltpu.sync_copy(idx_hbm.at[pl.ds(wid * chunks_per_w, chunks_per_w)], idx_v)
        for j in range(chunks_per_w):
            pltpu.async_copy(
                table_hbm.at[idx_v.at[j]],
                rows_v.at[pl.ds(j * idx_chunk, idx_chunk)],
                sem,
            ).wait()
        pltpu.sync_copy(rows_v, out_hbm.at[pl.ds(wid * cpw, cpw)])

    return k


def _bag_kernel(mp, acc_rows, gi):
    cpw = mp // NW
    ng = cpw // gi
    rows_per_g = gi * TOP_K
    acc_rows_out = 2048
    stripe = acc_rows_out // NS
    mesh = plsc.VectorSubcoreMesh(core_axis_name="c", subcore_axis_name="s")

    scratch = [
        pltpu.VMEM((cpw * TOP_K,), jnp.int32),
        pltpu.VMEM((rows_per_g, FEAT), jnp.float32),
        pltpu.VMEM((gi, FEAT), jnp.float32),
        pltpu.VMEM((ng, gi), jnp.int32),
        pltpu.VMEM_SHARED((acc_rows, FEAT), jnp.float32),
        pltpu.SemaphoreType.DMA,
    ]

    @functools.partial(
        pl.kernel,
        out_type=jax.ShapeDtypeStruct((NC, acc_rows_out, FEAT), jnp.float32),
        mesh=mesh,
        scratch_types=scratch,
        compiler_params=pltpu.CompilerParams(use_tc_tiling_on_sc=False),
    )
    def k(z_hbm, nbrs_hbm, dst_hbm, zeros_hbm, out_hbm, idx_v, rows_v, vec_v,
          dst_v, acc_sh, sem):
        cid = lax.axis_index("c")
        sid = lax.axis_index("s")
        wid = sid * NC + cid
        base = wid * cpw

        @pl.when(sid == 0)
        def _zero():
            pltpu.sync_copy(zeros_hbm, acc_sh.at[pl.ds(0, acc_rows_out)])

        pltpu.sync_copy(nbrs_hbm.at[pl.ds(base * TOP_K, cpw * TOP_K)], idx_v)
        pltpu.sync_copy(dst_hbm.at[pl.ds(wid * ng, ng)], dst_v)
        plsc.subcore_barrier()

        def group(g, _):
            pltpu.async_copy(
                z_hbm.at[idx_v.at[pl.ds(g * rows_per_g, rows_per_g)]],
                rows_v, sem,
            ).wait()
            for i in range(gi):
                for f in range(FEAT // 16):
                    s = rows_v[i * TOP_K, pl.ds(f * 16, 16)]
                    for j in range(1, TOP_K):
                        s = s + rows_v[i * TOP_K + j, pl.ds(f * 16, 16)]
                    vec_v[i, pl.ds(f * 16, 16)] = s
            pltpu.sync_copy(vec_v, acc_sh.at[dst_v.at[g]], add=True)
            return 0

        lax.fori_loop(0, ng, group, 0)
        plsc.subcore_barrier()
        pltpu.sync_copy(
            acc_sh.at[pl.ds(sid * stripe, stripe)],
            out_hbm.at[cid].at[pl.ds(sid * stripe, stripe)],
        )

    return k


def _finish_body(p_ref, su_ref, wi_ref, bi_ref, o_ref):
    acc = p_ref[0] + p_ref[1]
    item = acc[:1024] * (1.0 / TOP_K)
    user = acc[1024:2048] * su_ref[...]
    x = item * user
    logits = jnp.dot(x, wi_ref[...], preferred_element_type=jnp.float32) + bi_ref[0, 0]
    o_ref[...] = jax.nn.sigmoid(logits)


def _finish(partial, scale_user, W_int, b_int):
    return pl.pallas_call(
        _finish_body,
        in_specs=[
            pl.BlockSpec(partial.shape, lambda: (0, 0, 0)),
            pl.BlockSpec((1024, 1), lambda: (0, 0)),
            pl.BlockSpec((FEAT, 1), lambda: (0, 0)),
            pl.BlockSpec((1, 1), lambda: (0, 0)),
        ],
        out_specs=pl.BlockSpec((1024, 1), lambda: (0, 0)),
        out_shape=jax.ShapeDtypeStruct((1024, 1), jnp.float32),
    )(partial, scale_user, W_int, b_int)


def kernel(item_idxs, user_items_flat, cu_seqlens, neighbor_table, feat_tables,
           W1a, b1a, W1b, b1b, W2, b2, W_int, b_int):
    B = item_idxs.shape[0]
    T = user_items_flat.shape[0]
    M = B + T
    mp = ((M + 8 * NW - 1) // (8 * NW)) * (8 * NW)
    cpw = mp // NW
    acc_rows = 2 * B + 8

    item_idxs = item_idxs.astype(jnp.int32)
    user_items_flat = user_items_flat.astype(jnp.int32)
    cu_seqlens = cu_seqlens.astype(jnp.int32)
    neighbor_table = neighbor_table.astype(jnp.int32)

    z = _build_z(feat_tables, W1a, b1a, W1b, b1b, W2, b2)

    all_idx = jnp.concatenate(
        [item_idxs, user_items_flat,
         jnp.zeros((mp - M,), jnp.int32)])
    seg_ids = jnp.searchsorted(
        cu_seqlens[1:], jnp.arange(T, dtype=jnp.int32), side='right'
    ).astype(jnp.int32)
    dst = jnp.concatenate(
        [jnp.arange(B, dtype=jnp.int32), B + seg_ids,
         jnp.full((mp - M,), 2 * B, jnp.int32)])

    idx_chunk = 74 if cpw % 74 == 0 else 8
    while cpw % idx_chunk != 0 or idx_chunk > 128:
        idx_chunk //= 2
    chunks_per_w = cpw // idx_chunk
    idx2 = all_idx.reshape(mp // idx_chunk, idx_chunk)
    nbrs = _nbr_gather_kernel(mp, idx_chunk, chunks_per_w)(neighbor_table, idx2)

    zeros = jnp.zeros((2048, FEAT), jnp.float32)
    gi = 8
    partial = _bag_kernel(mp, acc_rows, gi=gi)(
        z, nbrs.reshape(mp * TOP_K), dst.reshape(mp // gi, gi), zeros)

    counts = jnp.diff(cu_seqlens).astype(jnp.float32)
    scale_user = (1.0 / (TOP_K * jnp.maximum(counts, 1.0))).reshape(B, 1)
    pred = _finish(partial, scale_user, W_int.reshape(FEAT, 1),
                   b_int.reshape(1, 1))
    return pred.reshape(-1)


# speedup vs baseline: 2.7562x; 2.5277x over previous
import functools

import jax
import jax.numpy as jnp
from jax import lax
from jax.experimental import pallas as pl
from jax.experimental.pallas import tpu as pltpu
from jax.experimental.pallas import tpu_sc as plsc

N_ITEMS = 100000
IN_DIM = 64
OUT_DIM = 64
N_COM = 5
TOP_K = 8
FEAT = N_COM * OUT_DIM

NC = 2
NS = 16
NW = NC * NS

_Z_ROWS_BLK = 1024
_DN_T = (((0,), (0,)), ((), ()))


def _zbuild_body(ft_ref, w1a_ref, b1a_ref, w1b_ref, b1b_ref, w2_ref, b2_ref, z_ref):
    for c in range(N_COM):
        xa_t = ft_ref[2 * c]
        xb_t = ft_ref[2 * c + 1]
        ha = jnp.maximum(
            lax.dot_general(xa_t, w1a_ref[c], _DN_T,
                            preferred_element_type=jnp.float32) + b1a_ref[c], 0.0)
        hb = jnp.maximum(
            lax.dot_general(xb_t, w1b_ref[c], _DN_T,
                            preferred_element_type=jnp.float32) + b1b_ref[c], 0.0)
        z = jnp.dot(ha + hb, w2_ref[c], preferred_element_type=jnp.float32) + b2_ref[c]
        z_ref[:, c * OUT_DIM:(c + 1) * OUT_DIM] = z


def _build_z(ft_t, W1a, b1a, W1b, b1b, W2, b2):
    n = ft_t.shape[2]
    grid = (n + _Z_ROWS_BLK - 1) // _Z_ROWS_BLK
    return pl.pallas_call(
        _zbuild_body,
        grid=(grid,),
        in_specs=[
            pl.BlockSpec((2 * N_COM, IN_DIM, _Z_ROWS_BLK), lambda i: (0, 0, i)),
            pl.BlockSpec((N_COM, IN_DIM, OUT_DIM), lambda i: (0, 0, 0)),
            pl.BlockSpec((N_COM, OUT_DIM), lambda i: (0, 0)),
            pl.BlockSpec((N_COM, IN_DIM, OUT_DIM), lambda i: (0, 0, 0)),
            pl.BlockSpec((N_COM, OUT_DIM), lambda i: (0, 0)),
            pl.BlockSpec((N_COM, OUT_DIM, OUT_DIM), lambda i: (0, 0, 0)),
            pl.BlockSpec((N_COM, OUT_DIM), lambda i: (0, 0)),
        ],
        out_specs=pl.BlockSpec((_Z_ROWS_BLK, FEAT), lambda i: (i, 0)),
        out_shape=jax.ShapeDtypeStruct((n, FEAT), jnp.float32),
    )(ft_t, W1a, b1a, W1b, b1b, W2, b2)


def _nbr_gather_kernel(mp, idx_chunk, chunks_per_w):
    cpw = mp // NW
    mesh = plsc.VectorSubcoreMesh(core_axis_name="c", subcore_axis_name="s")

    @functools.partial(
        pl.kernel,
        out_type=jax.ShapeDtypeStruct((mp, TOP_K), jnp.int32),
        mesh=mesh,
        scratch_types=[
            pltpu.VMEM((chunks_per_w, idx_chunk), jnp.int32),
            pltpu.VMEM((cpw, TOP_K), jnp.int32),
            pltpu.SemaphoreType.DMA,
        ],
        compiler_params=pltpu.CompilerParams(use_tc_tiling_on_sc=False),
    )
    def k(table_hbm, idx_hbm, out_hbm, idx_v, rows_v, sem):
        wid = lax.axis_index("s") * NC + lax.axis_index("c")
        pltpu.sync_copy(idx_hbm.at[pl.ds(wid * chunks_per_w, chunks_per_w)], idx_v)
        for j in range(chunks_per_w):
            pltpu.async_copy(
                table_hbm.at[idx_v.at[j]],
                rows_v.at[pl.ds(j * idx_chunk, idx_chunk)],
                sem,
            ).wait()
        pltpu.sync_copy(rows_v, out_hbm.at[pl.ds(wid * cpw, cpw)])

    return k


def _bag_kernel(mp, acc_rows, gi):
    cpw = mp // NW
    ng = cpw // gi
    acc_rows_out = 2048
    stripe = acc_rows_out // NS
    mesh = plsc.VectorSubcoreMesh(core_axis_name="c", subcore_axis_name="s")

    scratch = [
        pltpu.VMEM((TOP_K, ng, gi), jnp.int32),
        pltpu.VMEM((gi, FEAT), jnp.float32),
        pltpu.VMEM((ng, gi), jnp.int32),
        pltpu.VMEM_SHARED((acc_rows, FEAT), jnp.float32),
        pltpu.SemaphoreType.DMA,
    ]

    @functools.partial(
        pl.kernel,
        out_type=jax.ShapeDtypeStruct((NC, acc_rows_out, FEAT), jnp.float32),
        mesh=mesh,
        scratch_types=scratch,
        compiler_params=pltpu.CompilerParams(use_tc_tiling_on_sc=False),
    )
    def k(z_hbm, nbrs_hbm, dst_hbm, zeros_hbm, out_hbm, idx_v, vec_v, dst_v,
          acc_sh, sem):
        cid = lax.axis_index("c")
        sid = lax.axis_index("s")
        wid = sid * NC + cid

        @pl.when(sid == 0)
        def _zero():
            pltpu.sync_copy(zeros_hbm, acc_sh.at[pl.ds(0, acc_rows_out)])

        pltpu.sync_copy(nbrs_hbm.at[wid], idx_v)
        pltpu.sync_copy(dst_hbm.at[wid], dst_v)
        plsc.subcore_barrier()

        def group(g, _):
            pltpu.sync_copy(zeros_hbm.at[pl.ds(0, gi)], vec_v)
            cps = [
                pltpu.async_copy(z_hbm.at[idx_v.at[kk].at[g]], vec_v, sem,
                                 add=True)
                for kk in range(TOP_K)
            ]
            for c in cps:
                c.wait()
            pltpu.sync_copy(vec_v, acc_sh.at[dst_v.at[g]], add=True)
            return 0

        lax.fori_loop(0, ng, group, 0)
        plsc.subcore_barrier()
        pltpu.sync_copy(
            acc_sh.at[pl.ds(sid * stripe, stripe)],
            out_hbm.at[cid].at[pl.ds(sid * stripe, stripe)],
        )

    return k


def _finish_body(p_ref, su_ref, wi_ref, bi_ref, o_ref):
    acc = p_ref[0] + p_ref[1]
    item = acc[:1024] * (1.0 / TOP_K)
    user = acc[1024:2048] * su_ref[...]
    x = item * user
    logits = jnp.dot(x, wi_ref[...], preferred_element_type=jnp.float32) + bi_ref[0, 0]
    o_ref[...] = jax.nn.sigmoid(logits)


def _finish(partial, scale_user, W_int, b_int):
    return pl.pallas_call(
        _finish_body,
        in_specs=[
            pl.BlockSpec(partial.shape, lambda: (0, 0, 0)),
            pl.BlockSpec((1024, 1), lambda: (0, 0)),
            pl.BlockSpec((FEAT, 1), lambda: (0, 0)),
            pl.BlockSpec((1, 1), lambda: (0, 0)),
        ],
        out_specs=pl.BlockSpec((1024, 1), lambda: (0, 0)),
        out_shape=jax.ShapeDtypeStruct((1024, 1), jnp.float32),
    )(partial, scale_user, W_int, b_int)


def kernel(item_idxs, user_items_flat, cu_seqlens, neighbor_table, feat_tables,
           W1a, b1a, W1b, b1b, W2, b2, W_int, b_int):
    B = item_idxs.shape[0]
    T = user_items_flat.shape[0]
    M = B + T
    mp = ((M + 8 * NW - 1) // (8 * NW)) * (8 * NW)
    cpw = mp // NW
    acc_rows = 2 * B + 8

    item_idxs = item_idxs.astype(jnp.int32)
    user_items_flat = user_items_flat.astype(jnp.int32)
    cu_seqlens = cu_seqlens.astype(jnp.int32)
    neighbor_table = neighbor_table.astype(jnp.int32)

    ft_t = jnp.transpose(feat_tables, (0, 2, 1))
    z = _build_z(ft_t, W1a, b1a, W1b, b1b, W2, b2)

    all_idx = jnp.concatenate(
        [item_idxs, user_items_flat,
         jnp.zeros((mp - M,), jnp.int32)])
    seg_ids = jnp.cumsum(
        jnp.zeros((T,), jnp.int32).at[cu_seqlens[1:-1]].add(1))
    dst = jnp.concatenate(
        [jnp.arange(B, dtype=jnp.int32), B + seg_ids,
         jnp.full((mp - M,), 2 * B, jnp.int32)])

    idx_chunk = 74 if cpw % 74 == 0 else 8
    while cpw % idx_chunk != 0 or idx_chunk > 128:
        idx_chunk //= 2
    chunks_per_w = cpw // idx_chunk
    idx2 = all_idx.reshape(mp // idx_chunk, idx_chunk)
    nbrs = _nbr_gather_kernel(mp, idx_chunk, chunks_per_w)(neighbor_table, idx2)

    gi = 37 if cpw % 37 == 0 else 8
    ng = cpw // gi
    nbrs4 = nbrs.reshape(NW, cpw, TOP_K).transpose(0, 2, 1).reshape(
        NW, TOP_K, ng, gi)
    dst3 = dst.reshape(NW, ng, gi)
    zeros = jnp.zeros((2048, FEAT), jnp.float32)
    partial = _bag_kernel(mp, acc_rows, gi=gi)(z, nbrs4, dst3, zeros)

    counts = jnp.diff(cu_seqlens).astype(jnp.float32)
    scale_user = (1.0 / (TOP_K * jnp.maximum(counts, 1.0))).reshape(B, 1)
    pred = _finish(partial, scale_user, W_int.reshape(FEAT, 1),
                   b_int.reshape(1, 1))
    return pred.reshape(-1)


# speedup vs baseline: 3.4125x; 1.2381x over previous
import functools

import jax
import jax.numpy as jnp
from jax import lax
from jax.experimental import pallas as pl
from jax.experimental.pallas import tpu as pltpu
from jax.experimental.pallas import tpu_sc as plsc

N_ITEMS = 100000
IN_DIM = 64
OUT_DIM = 64
N_COM = 5
TOP_K = 8
FEAT = N_COM * OUT_DIM
NPL = 3

NC = 2
NS = 16
NW = NC * NS

_Z_ROWS_BLK = 1024
_DN_T = (((0,), (0,)), ((), ()))


def _zbuild_body(ft_ref, w1a_ref, b1a_ref, w1b_ref, b1b_ref, w2_ref, b2_ref, z_ref):
    zero = jnp.zeros((_Z_ROWS_BLK, NPL * 128 - FEAT), jnp.float32)
    cols = []
    for c in range(N_COM):
        xa_t = ft_ref[2 * c]
        xb_t = ft_ref[2 * c + 1]
        ha = jnp.maximum(
            lax.dot_general(xa_t, w1a_ref[c], _DN_T,
                            preferred_element_type=jnp.float32) + b1a_ref[c], 0.0)
        hb = jnp.maximum(
            lax.dot_general(xb_t, w1b_ref[c], _DN_T,
                            preferred_element_type=jnp.float32) + b1b_ref[c], 0.0)
        cols.append(
            jnp.dot(ha + hb, w2_ref[c], preferred_element_type=jnp.float32)
            + b2_ref[c])
    z = jnp.concatenate(cols + [zero], axis=1)
    for j in range(NPL):
        z_ref[j] = z[:, j * 128:(j + 1) * 128]


def _build_z(ft_t, W1a, b1a, W1b, b1b, W2, b2):
    n = ft_t.shape[2]
    grid = (n + _Z_ROWS_BLK - 1) // _Z_ROWS_BLK
    return pl.pallas_call(
        _zbuild_body,
        grid=(grid,),
        in_specs=[
            pl.BlockSpec((2 * N_COM, IN_DIM, _Z_ROWS_BLK), lambda i: (0, 0, i)),
            pl.BlockSpec((N_COM, IN_DIM, OUT_DIM), lambda i: (0, 0, 0)),
            pl.BlockSpec((N_COM, OUT_DIM), lambda i: (0, 0)),
            pl.BlockSpec((N_COM, IN_DIM, OUT_DIM), lambda i: (0, 0, 0)),
            pl.BlockSpec((N_COM, OUT_DIM), lambda i: (0, 0)),
            pl.BlockSpec((N_COM, OUT_DIM, OUT_DIM), lambda i: (0, 0, 0)),
            pl.BlockSpec((N_COM, OUT_DIM), lambda i: (0, 0)),
        ],
        out_specs=pl.BlockSpec((NPL, _Z_ROWS_BLK, 128), lambda i: (0, i, 0)),
        out_shape=jax.ShapeDtypeStruct((NPL, n, 128), jnp.float32),
    )(ft_t, W1a, b1a, W1b, b1b, W2, b2)


def _bag_kernel(mp, acc_rows, gi, ic):
    cpw = mp // NW
    ng = cpw // gi
    nic = cpw // ic
    acc_rows_out = 2048
    stripe = acc_rows_out // NS
    mesh = plsc.VectorSubcoreMesh(core_axis_name="c", subcore_axis_name="s")

    scratch = [
        pltpu.VMEM((TOP_K, cpw), jnp.int32),
        pltpu.VMEM((TOP_K, cpw), jnp.int32),
        pltpu.VMEM((NPL, gi, 128), jnp.float32),
        pltpu.VMEM((ng, gi), jnp.int32),
        pltpu.VMEM_SHARED((NPL, acc_rows, 128), jnp.float32),
        pltpu.SemaphoreType.DMA,
    ]

    @functools.partial(
        pl.kernel,
        out_type=jax.ShapeDtypeStruct((NC, NPL, acc_rows_out, 128), jnp.float32),
        mesh=mesh,
        scratch_types=scratch,
        compiler_params=pltpu.CompilerParams(use_tc_tiling_on_sc=False),
    )
    def k(z_hbm, nbrflat_hbm, keys_hbm, dst_hbm, zeros_hbm, out_hbm,
          keys_v, idx_v, vec_v, dst_v, acc_sh, sem):
        cid = lax.axis_index("c")
        sid = lax.axis_index("s")
        wid = sid * NC + cid

        @pl.when(sid == 0)
        def _zero():
            for j in range(NPL):
                pltpu.sync_copy(zeros_hbm, acc_sh.at[j].at[pl.ds(0, acc_rows_out)])

        pltpu.sync_copy(keys_hbm.at[wid], keys_v)
        pltpu.sync_copy(dst_hbm.at[wid], dst_v)

        cps = [
            pltpu.async_copy(
                nbrflat_hbm.at[keys_v.at[kk].at[pl.ds(j * ic, ic)]],
                idx_v.at[kk].at[pl.ds(j * ic, ic)],
                sem,
            )
            for kk in range(TOP_K)
            for j in range(nic)
        ]
        for c in cps:
            c.wait()
        plsc.subcore_barrier()

        def group(g, _):
            for j in range(NPL):
                pltpu.sync_copy(zeros_hbm.at[pl.ds(0, gi)], vec_v.at[j])
            adds = [
                pltpu.async_copy(
                    z_hbm.at[j].at[idx_v.at[kk].at[pl.ds(g * gi, gi)]],
                    vec_v.at[j], sem, add=True)
                for kk in range(TOP_K)
                for j in range(NPL)
            ]
            for c in adds:
                c.wait()
            for j in range(NPL):
                pltpu.sync_copy(vec_v.at[j], acc_sh.at[j].at[dst_v.at[g]],
                                add=True)
            return 0

        lax.fori_loop(0, ng, group, 0)
        plsc.subcore_barrier()
        for j in range(NPL):
            pltpu.sync_copy(
                acc_sh.at[j].at[pl.ds(sid * stripe, stripe)],
                out_hbm.at[cid].at[j].at[pl.ds(sid * stripe, stripe)],
            )

    return k


def _finish_body(p_ref, su_ref, wi_ref, bi_ref, o_ref):
    logits = bi_ref[0, 0]
    for j in range(NPL):
        acc = p_ref[0, j] + p_ref[1, j]
        item = acc[:1024] * (1.0 / TOP_K)
        user = acc[1024:2048] * su_ref[...]
        logits = logits + jnp.dot(item * user, wi_ref[j],
                                  preferred_element_type=jnp.float32)
    o_ref[...] = jax.nn.sigmoid(logits)


def _finish(partial, scale_user, W_int3, b_int):
    return pl.pallas_call(
        _finish_body,
        in_specs=[
            pl.BlockSpec(partial.shape, lambda: (0, 0, 0, 0)),
            pl.BlockSpec((1024, 1), lambda: (0, 0)),
            pl.BlockSpec((NPL, 128, 1), lambda: (0, 0, 0)),
            pl.BlockSpec((1, 1), lambda: (0, 0)),
        ],
        out_specs=pl.BlockSpec((1024, 1), lambda: (0, 0)),
        out_shape=jax.ShapeDtypeStruct((1024, 1), jnp.float32),
    )(partial, scale_user, W_int3, b_int)


def kernel(item_idxs, user_items_flat, cu_seqlens, neighbor_table, feat_tables,
           W1a, b1a, W1b, b1b, W2, b2, W_int, b_int):
    B = item_idxs.shape[0]
    T = user_items_flat.shape[0]
    n_items = neighbor_table.shape[0]
    M = B + T
    gi = 80
    ic = 80
    cpw = ((M + NW * gi - 1) // (NW * gi)) * gi
    mp = cpw * NW
    acc_rows = 2 * B + 8

    item_idxs = item_idxs.astype(jnp.int32)
    user_items_flat = user_items_flat.astype(jnp.int32)
    cu_seqlens = cu_seqlens.astype(jnp.int32)
    neighbor_table = neighbor_table.astype(jnp.int32)

    ft_t = jnp.transpose(feat_tables, (0, 2, 1))
    z3 = _build_z(ft_t, W1a, b1a, W1b, b1b, W2, b2)

    all_idx = jnp.concatenate(
        [item_idxs, user_items_flat,
         jnp.zeros((mp - M,), jnp.int32)])
    seg_ids = jnp.cumsum(
        jnp.zeros((T,), jnp.int32).at[cu_seqlens[1:-1]].add(1))
    dst = jnp.concatenate(
        [jnp.arange(B, dtype=jnp.int32), B + seg_ids,
         jnp.full((mp - M,), 2 * B, jnp.int32)])
    dst3 = dst.reshape(NW, cpw // gi, gi)

    nbr_flat = jnp.transpose(neighbor_table).reshape(-1)
    keys = (all_idx.reshape(NW, 1, cpw)
            + (n_items * jnp.arange(TOP_K, dtype=jnp.int32)).reshape(1, TOP_K, 1))

    zeros = jnp.zeros((2048, 128), jnp.float32)
    partial = _bag_kernel(mp, acc_rows, gi=gi, ic=ic)(
        z3, nbr_flat, keys, dst3, zeros)

    counts = jnp.diff(cu_seqlens).astype(jnp.float32)
    scale_user = (1.0 / (TOP_K * jnp.maximum(counts, 1.0))).reshape(B, 1)
    W_int3 = jnp.concatenate(
        [W_int.reshape(FEAT, 1),
         jnp.zeros((NPL * 128 - FEAT, 1), jnp.float32)]).reshape(NPL, 128, 1)
    pred = _finish(partial, scale_user, W_int3, b_int.reshape(1, 1))
    return pred.reshape(-1)
